# baseline (speedup 1.0000x reference)
import functools

import jax
import jax.numpy as jnp
from jax import lax
from jax.experimental import pallas as pl
from jax.experimental.pallas import tpu as pltpu
from jax.experimental.pallas import tpu_sc as plsc

VOCAB = 119
FEAT = 200
D_MODEL = 512
L_SEQ = 20
VPAD = 128

NC, NS = 2, 16
NW = NC * NS
BPW = 16384 // NW
NB = 2
NCH = BPW // NB
L_MAIN = 16
L_TAIL = 8


def _proj_body(cbfv_ref, w_ref, b_ref, out_ref):
    out_ref[...] = (
        jnp.dot(cbfv_ref[...], w_ref[...], preferred_element_type=jnp.float32)
        + b_ref[...]
    )


def _compute_proj(cbfv_pad, W, b_row):
    return pl.pallas_call(
        _proj_body,
        out_shape=jax.ShapeDtypeStruct((VPAD, D_MODEL), jnp.float32),
    )(cbfv_pad, W, b_row)


_SC_MESH = plsc.VectorSubcoreMesh(
    core_axis_name="c", subcore_axis_name="s", num_cores=NC, num_subcores=NS
)


@functools.partial(
    pl.kernel,
    out_type=jax.ShapeDtypeStruct((16384, L_SEQ, D_MODEL), jnp.float32),
    mesh=_SC_MESH,
    scratch_types=[
        pltpu.VMEM((BPW * L_MAIN,), jnp.int32),
        pltpu.VMEM((BPW * L_TAIL,), jnp.int32),
        pltpu.VMEM((NB, L_SEQ, D_MODEL), jnp.float32),
        pltpu.VMEM((NB, L_SEQ, D_MODEL), jnp.float32),
        pltpu.VMEM((NB * L_TAIL, D_MODEL), jnp.float32),
        pltpu.VMEM((NB * L_TAIL, D_MODEL), jnp.float32),
        pltpu.SemaphoreType.DMA,
        pltpu.SemaphoreType.DMA,
        pltpu.SemaphoreType.DMA,
        pltpu.SemaphoreType.DMA,
    ],
)
def _sc_gather(
    idxm_hbm, idxt_hbm, tab_hbm, out_hbm,
    idxm_v, idxt_v, rows0, rows1, tail0, tail1, g0, g1, w0, w1,
):
    wid = lax.axis_index("s") * NC + lax.axis_index("c")
    b_base = wid * BPW
    pltpu.sync_copy(idxm_hbm.at[pl.ds(b_base * L_MAIN, BPW * L_MAIN)], idxm_v)
    pltpu.sync_copy(idxt_hbm.at[pl.ds(b_base * L_TAIL, BPW * L_TAIL)], idxt_v)

    bufs = (rows0, rows1)
    tails = (tail0, tail1)
    gsem = (g0, g1)
    wsem = (w0, w1)

    def gather_start(i, b):
        for j in range(NB):
            pltpu.async_copy(
                tab_hbm.at[idxm_v.at[pl.ds((i * NB + j) * L_MAIN, L_MAIN)]],
                bufs[b].at[j, pl.ds(0, L_MAIN)],
                gsem[b],
            )
        pltpu.async_copy(
            tab_hbm.at[idxt_v.at[pl.ds(i * NB * L_TAIL, NB * L_TAIL)]],
            tails[b],
            gsem[b],
        )

    def gather_wait(i, b):
        for j in range(NB):
            pltpu.make_async_copy(
                tab_hbm.at[idxm_v.at[pl.ds((i * NB + j) * L_MAIN, L_MAIN)]],
                bufs[b].at[j, pl.ds(0, L_MAIN)],
                gsem[b],
            ).wait()
        pltpu.make_async_copy(
            tab_hbm.at[idxt_v.at[pl.ds(i * NB * L_TAIL, NB * L_TAIL)]],
            tails[b],
            gsem[b],
        ).wait()

    def tail_fix(b):
        for j in range(NB):
            for r in range(L_SEQ - L_MAIN):
                for c in range(D_MODEL // 16):
                    bufs[b][j, L_MAIN + r, pl.ds(16 * c, 16)] = tails[b][
                        j * L_TAIL + r, pl.ds(16 * c, 16)
                    ]

    def write_start(i, b):
        pltpu.async_copy(bufs[b], out_hbm.at[pl.ds(b_base + i * NB, NB)], wsem[b])

    def write_wait(i, b):
        pltpu.make_async_copy(
            bufs[b], out_hbm.at[pl.ds(b_base + i * NB, NB)], wsem[b]
        ).wait()

    gather_start(0, 0)
    gather_wait(0, 0)
    tail_fix(0)
    write_start(0, 0)
    gather_start(1, 1)

    def pair_step(k, carry):
        i1 = 1 + 2 * k
        gather_wait(i1, 1)
        tail_fix(1)
        write_start(i1, 1)
        write_wait(i1 - 1, 0)
        gather_start(i1 + 1, 0)
        i2 = i1 + 1
        gather_wait(i2, 0)
        tail_fix(0)
        write_start(i2, 0)
        write_wait(i2 - 1, 1)
        gather_start(i2 + 1, 1)
        return carry

    lax.fori_loop(0, (NCH - 2) // 2, pair_step, 0)

    last = NCH - 1
    gather_wait(last, 1)
    tail_fix(1)
    write_start(last, 1)
    write_wait(last - 1, 0)
    write_wait(last, 1)


def kernel(src, cbfv, W, b):
    cbfv_pad = jnp.pad(cbfv, ((0, VPAD - VOCAB), (0, 0)))
    proj = _compute_proj(cbfv_pad, W, b.reshape(1, D_MODEL))
    src32 = src.astype(jnp.int32)
    idx_main = src32[:, :L_MAIN].reshape(-1)
    idx_tail = jnp.pad(src32[:, L_MAIN:], ((0, 0), (0, L_TAIL - 4))).reshape(-1)
    return _sc_gather(idx_main, idx_tail, proj)


# speedup vs baseline: 3.2561x; 3.2561x over previous
import functools

import jax
import jax.numpy as jnp
from jax import lax
from jax.experimental import pallas as pl
from jax.experimental.pallas import tpu as pltpu
from jax.experimental.pallas import tpu_sc as plsc

VOCAB = 119
FEAT = 200
D_MODEL = 512
L_SEQ = 20
VPAD = 128

NC, NS = 2, 16
NW = NC * NS
BPW = 16384 // NW
NB = 2
NCH = BPW // NB
L_MAIN = 16
L_TAIL = L_SEQ - L_MAIN
LANES = 16


def _proj_body(cbfv_ref, w_ref, b_ref, out_ref):
    out_ref[...] = (
        jnp.dot(cbfv_ref[...], w_ref[...], preferred_element_type=jnp.float32)
        + b_ref[...]
    )


def _compute_proj(cbfv_pad, W, b_row):
    return pl.pallas_call(
        _proj_body,
        out_shape=jax.ShapeDtypeStruct((VPAD, D_MODEL), jnp.float32),
    )(cbfv_pad, W, b_row)


_SC_MESH = plsc.VectorSubcoreMesh(
    core_axis_name="c", subcore_axis_name="s", num_cores=NC, num_subcores=NS
)


@functools.partial(
    pl.kernel,
    out_type=jax.ShapeDtypeStruct((16384, L_SEQ, D_MODEL), jnp.float32),
    mesh=_SC_MESH,
    compiler_params=pltpu.CompilerParams(needs_layout_passes=False),
    scratch_types=[
        pltpu.VMEM((BPW * L_MAIN,), jnp.int32),
        pltpu.VMEM((BPW * L_TAIL + LANES,), jnp.int32),
        pltpu.VMEM((VPAD * D_MODEL,), jnp.float32),
        pltpu.VMEM((NB, L_SEQ, D_MODEL), jnp.float32),
        pltpu.VMEM((NB, L_SEQ, D_MODEL), jnp.float32),
        pltpu.SemaphoreType.DMA,
        pltpu.SemaphoreType.DMA,
        pltpu.SemaphoreType.DMA,
        pltpu.SemaphoreType.DMA,
    ],
)
def _sc_gather(
    idxm_hbm, idxt_hbm, tab_hbm, tabflat_hbm, out_hbm,
    idxm_v, idxt_v, tab_v, rows0, rows1, g0, g1, w0, w1,
):
    wid = lax.axis_index("s") * NC + lax.axis_index("c")
    b_base = wid * BPW
    pltpu.sync_copy(idxm_hbm.at[pl.ds(b_base * L_MAIN, BPW * L_MAIN)], idxm_v)
    pltpu.sync_copy(
        idxt_hbm.at[pl.ds(b_base * L_TAIL, BPW * L_TAIL)],
        idxt_v.at[pl.ds(0, BPW * L_TAIL)],
    )
    pltpu.sync_copy(tabflat_hbm, tab_v)

    bufs = (rows0, rows1)
    gsem = (g0, g1)
    wsem = (w0, w1)
    lane_iota = lax.iota(jnp.int32, LANES)

    def gather_start(i, b):
        for j in range(NB):
            pltpu.async_copy(
                tab_hbm.at[idxm_v.at[pl.ds((i * NB + j) * L_MAIN, L_MAIN)]],
                bufs[b].at[j, pl.ds(0, L_MAIN)],
                gsem[b],
            )

    def gather_wait(i, b):
        for j in range(NB):
            pltpu.make_async_copy(
                tab_hbm.at[idxm_v.at[pl.ds((i * NB + j) * L_MAIN, L_MAIN)]],
                bufs[b].at[j, pl.ds(0, L_MAIN)],
                gsem[b],
            ).wait()

    def tail_fill(i, b):
        tvec = idxt_v[pl.ds(i * (NB * L_TAIL), LANES)]
        for j in range(NB):
            for r in range(L_TAIL):
                sel = jnp.full((LANES,), j * L_TAIL + r, dtype=jnp.int32)
                rvec = tvec.at[sel].get(mode="promise_in_bounds")
                base_word = rvec * D_MODEL + lane_iota
                for c in range(D_MODEL // LANES):
                    vals = plsc.load_gather(tab_v, [base_word + LANES * c])
                    bufs[b][j, L_MAIN + r, pl.ds(LANES * c, LANES)] = vals

    def write_start(i, b):
        pltpu.async_copy(bufs[b], out_hbm.at[pl.ds(b_base + i * NB, NB)], wsem[b])

    def write_wait(i, b):
        pltpu.make_async_copy(
            bufs[b], out_hbm.at[pl.ds(b_base + i * NB, NB)], wsem[b]
        ).wait()

    gather_start(0, 0)
    tail_fill(0, 0)
    gather_wait(0, 0)
    write_start(0, 0)
    gather_start(1, 1)

    def pair_step(k, carry):
        i1 = 1 + 2 * k
        tail_fill(i1, 1)
        gather_wait(i1, 1)
        write_start(i1, 1)
        write_wait(i1 - 1, 0)
        gather_start(i1 + 1, 0)
        i2 = i1 + 1
        tail_fill(i2, 0)
        gather_wait(i2, 0)
        write_start(i2, 0)
        write_wait(i2 - 1, 1)
        gather_start(i2 + 1, 1)
        return carry

    lax.fori_loop(0, (NCH - 2) // 2, pair_step, 0)

    last = NCH - 1
    tail_fill(last, 1)
    gather_wait(last, 1)
    write_start(last, 1)
    write_wait(last - 1, 0)
    write_wait(last, 1)


def kernel(src, cbfv, W, b):
    cbfv_pad = jnp.pad(cbfv, ((0, VPAD - VOCAB), (0, 0)))
    proj = _compute_proj(cbfv_pad, W, b.reshape(1, D_MODEL))
    src32 = src.astype(jnp.int32)
    idx_main = src32[:, :L_MAIN].reshape(-1)
    idx_tail = src32[:, L_MAIN:].reshape(-1)
    return _sc_gather(idx_main, idx_tail, proj, proj.reshape(-1))
